# Initial kernel scaffold; baseline (speedup 1.0000x reference)
import jax
import jax.numpy as jnp
from jax.experimental import pallas as pl


def kernel(predicted, targets):
    raise NotImplementedError("write your pallas kernel here")


# speedup vs baseline: 5.2296x; 5.2296x over previous
import functools

import jax
import jax.numpy as jnp
from jax import lax
from jax.experimental import pallas as pl
from jax.experimental.pallas import tpu as pltpu
from jax.experimental.pallas import tpu_sc as plsc

B = 128
V = 100000
BC = 12800
NB = -(-V // BC)
ROWS_PER_SUBCORE = 4


def _sc_gather_kernel(pred_hbm, tgt_hbm, out_hbm, tgt_v, idx_v, val_v, sem):
    core = lax.axis_index("c")
    sub = lax.axis_index("s")
    wid = sub * 2 + core

    @pl.when(wid < 8)
    def _():
        pltpu.sync_copy(tgt_hbm.at[pl.ds(wid * 16, 16)], tgt_v)
        t = tgt_v[...]
        idx_v[...] = (wid * 16 + lax.iota(jnp.int32, 16)) * V + t
        pltpu.async_copy(pred_hbm.at[idx_v], val_v, sem).wait()
        pltpu.sync_copy(val_v, out_hbm.at[pl.ds(wid * 16, 16)])


def _sc_gather(predicted, targets):
    mesh = plsc.VectorSubcoreMesh(core_axis_name="c", subcore_axis_name="s")
    kfn = functools.partial(
        pl.kernel,
        mesh=mesh,
        out_type=jax.ShapeDtypeStruct((B,), jnp.float32),
        scratch_types=[
            pltpu.VMEM((16,), jnp.int32),
            pltpu.VMEM((16,), jnp.int32),
            pltpu.VMEM((16,), jnp.float32),
            pltpu.SemaphoreType.DMA,
        ],
    )(_sc_gather_kernel)
    return kfn(predicted.reshape(B * V), targets)


def _tc_count_kernel(pred_ref, tgt_ref, v_ref, out_ref, acc_ref):
    c = pl.program_id(0)

    @pl.when(c == 0)
    def _init():
        acc_ref[...] = jnp.zeros_like(acc_ref)

    x = pred_ref[...]
    v = v_ref[...]
    t = tgt_ref[...]
    col = c * BC + lax.broadcasted_iota(jnp.int32, (B, BC), 1)
    beats = ((x > v) | ((x == v) & (col < t))) & (col < V)
    acc_ref[...] += jnp.sum(beats.astype(jnp.float32), axis=1, keepdims=True)

    @pl.when(c == NB - 1)
    def _fini():
        miss = (acc_ref[...] >= 29.5).astype(jnp.float32)
        out_ref[...] = jnp.sum(miss, axis=0, keepdims=True) * (1.0 / B)


def _tc_count(predicted, targets2d, v2d):
    return pl.pallas_call(
        _tc_count_kernel,
        grid=(NB,),
        in_specs=[
            pl.BlockSpec((B, BC), lambda c: (0, c)),
            pl.BlockSpec((B, 1), lambda c: (0, 0)),
            pl.BlockSpec((B, 1), lambda c: (0, 0)),
        ],
        out_specs=pl.BlockSpec((1, 1), lambda c: (0, 0)),
        out_shape=jax.ShapeDtypeStruct((1, 1), jnp.float32),
        scratch_shapes=[pltpu.VMEM((B, 1), jnp.float32)],
    )(predicted, targets2d, v2d)


def kernel(predicted, targets):
    v = _sc_gather(predicted, targets)
    out = _tc_count(predicted, targets.reshape(B, 1), v.reshape(B, 1))
    return out[0, 0]


# speedup vs baseline: 11.0750x; 2.1178x over previous
import functools

import jax
import jax.numpy as jnp
from jax import lax
from jax.experimental import pallas as pl
from jax.experimental.pallas import tpu as pltpu
from jax.experimental.pallas import tpu_sc as plsc

B = 128
V = 100000
BC = 12800
NB = -(-V // BC)
ROWS_PER_SUBCORE = 4


def _sc_gather_kernel(pred_hbm, tgt_hbm, out_hbm, tgt_v, idx_v, val_v, sem):
    core = lax.axis_index("c")
    sub = lax.axis_index("s")
    wid = sub * 2 + core

    @pl.when(wid < 8)
    def _():
        pltpu.sync_copy(tgt_hbm.at[pl.ds(wid * 16, 16)], tgt_v)
        t = tgt_v[...]
        idx_v[...] = (wid * 16 + lax.iota(jnp.int32, 16)) * V + t
        pltpu.async_copy(pred_hbm.at[idx_v], val_v, sem).wait()
        pltpu.sync_copy(val_v, out_hbm.at[pl.ds(wid * 16, 16)])


def _sc_gather(predicted, targets):
    mesh = plsc.VectorSubcoreMesh(core_axis_name="c", subcore_axis_name="s")
    kfn = functools.partial(
        pl.kernel,
        mesh=mesh,
        out_type=jax.ShapeDtypeStruct((B,), jnp.float32),
        scratch_types=[
            pltpu.VMEM((16,), jnp.int32),
            pltpu.VMEM((16,), jnp.int32),
            pltpu.VMEM((16,), jnp.float32),
            pltpu.SemaphoreType.DMA,
        ],
    )(_sc_gather_kernel)
    return kfn(predicted.reshape(B * V), targets)


def _tc_count_kernel(pred_ref, tgt_ref, v_ref, out_ref, acc_ref):
    c = pl.program_id(0)

    @pl.when(c == 0)
    def _init():
        acc_ref[...] = jnp.zeros_like(acc_ref)

    x = pred_ref[...]
    v = v_ref[...]
    t = tgt_ref[...]
    col = c * BC + lax.broadcasted_iota(jnp.int32, (B, BC), 1)
    beats = ((x > v) | ((x == v) & (col < t))) & (col < V)
    acc_ref[...] += jnp.sum(beats.astype(jnp.float32), axis=1, keepdims=True)

    @pl.when(c == NB - 1)
    def _fini():
        miss = (acc_ref[...] >= 29.5).astype(jnp.float32)
        out_ref[...] = jnp.sum(miss, axis=0, keepdims=True) * (1.0 / B)


def _tc_count(predicted, targets2d, v2d):
    return pl.pallas_call(
        _tc_count_kernel,
        grid=(NB,),
        in_specs=[
            pl.BlockSpec((B, BC), lambda c: (0, c)),
            pl.BlockSpec((B, 1), lambda c: (0, 0)),
            pl.BlockSpec((B, 1), lambda c: (0, 0)),
        ],
        out_specs=pl.BlockSpec((1, 1), lambda c: (0, 0)),
        out_shape=jax.ShapeDtypeStruct((1, 1), jnp.float32),
        scratch_shapes=[pltpu.VMEM((B, 1), jnp.float32)],
    )(predicted, targets2d, v2d)


def kernel(predicted, targets):
    v = jnp.take_along_axis(predicted, targets[:, None], axis=1)
    out = _tc_count(predicted, targets.reshape(B, 1), v)
    return out[0, 0]
